# Initial kernel scaffold; baseline (speedup 1.0000x reference)
import jax
import jax.numpy as jnp
from jax.experimental import pallas as pl


def kernel(X, pos_embed_weight):
    raise NotImplementedError("write your pallas kernel here")


# speedup vs baseline: 1.9887x; 1.9887x over previous
import functools

import jax
import jax.numpy as jnp
from jax import lax
from jax.experimental import pallas as pl
from jax.experimental.pallas import tpu as pltpu
from jax.experimental.pallas import tpu_sc as plsc

NUM_CORES = 2
NUM_SUBCORES = 16
NUM_WORKERS = NUM_CORES * NUM_SUBCORES

B = 4 * 8192
D = 1024
ROWS_PER_WORKER = B // NUM_WORKERS

_mesh = plsc.VectorSubcoreMesh(core_axis_name="c", subcore_axis_name="s")


CHUNK = 32


@functools.partial(
    pl.kernel,
    mesh=_mesh,
    out_type=jax.ShapeDtypeStruct((B, D), jnp.float32),
    scratch_types=[
        pltpu.VMEM((ROWS_PER_WORKER,), jnp.int32),
        pltpu.VMEM((CHUNK, D), jnp.float32),
        pltpu.SemaphoreType.DMA,
    ],
)
def _gather_kernel(idx_hbm, table_hbm, out_hbm, idx_v, rows_v, sem):
    wid = lax.axis_index("s") * NUM_CORES + lax.axis_index("c")
    base = wid * ROWS_PER_WORKER
    pltpu.sync_copy(idx_hbm.at[pl.ds(base, ROWS_PER_WORKER)], idx_v)

    @pl.loop(0, ROWS_PER_WORKER, step=CHUNK)
    def _(g):
        pltpu.async_copy(
            table_hbm.at[idx_v.at[pl.ds(g, CHUNK)]], rows_v, sem
        ).wait()
        pltpu.sync_copy(rows_v, out_hbm.at[pl.ds(base + g, CHUNK)])


def kernel(X, pos_embed_weight):
    flat_idx = X.reshape(-1).astype(jnp.int32)
    out = _gather_kernel(flat_idx, pos_embed_weight)
    return out.reshape(X.shape + (D,))


# speedup vs baseline: 2.3760x; 1.1947x over previous
import functools

import jax
import jax.numpy as jnp
from jax import lax
from jax.experimental import pallas as pl
from jax.experimental.pallas import tpu as pltpu
from jax.experimental.pallas import tpu_sc as plsc

NUM_CORES = 2
NUM_SUBCORES = 16
NUM_WORKERS = NUM_CORES * NUM_SUBCORES

B = 4 * 8192
D = 1024
ROWS_PER_WORKER = B // NUM_WORKERS

CHUNK = 32
NBUF = 2
N_CHUNKS = ROWS_PER_WORKER // CHUNK

_mesh = plsc.VectorSubcoreMesh(core_axis_name="c", subcore_axis_name="s")


@functools.partial(
    pl.kernel,
    mesh=_mesh,
    out_type=jax.ShapeDtypeStruct((B, D), jnp.float32),
    scratch_types=[
        pltpu.VMEM((ROWS_PER_WORKER,), jnp.int32),
        pltpu.VMEM((NBUF, CHUNK, D), jnp.float32),
        pltpu.SemaphoreType.DMA((NBUF,)),
        pltpu.SemaphoreType.DMA((NBUF,)),
    ],
)
def _gather_kernel(idx_hbm, table_hbm, out_hbm, idx_v, rows_v, gsem, wsem):
    wid = lax.axis_index("s") * NUM_CORES + lax.axis_index("c")
    base = wid * ROWS_PER_WORKER
    pltpu.sync_copy(idx_hbm.at[pl.ds(base, ROWS_PER_WORKER)], idx_v)

    def start_gather(g, b):
        pltpu.async_copy(
            table_hbm.at[idx_v.at[pl.ds(g * CHUNK, CHUNK)]],
            rows_v.at[b],
            gsem.at[b],
        )

    def start_write(g, b):
        pltpu.async_copy(
            rows_v.at[b],
            out_hbm.at[pl.ds(base + g * CHUNK, CHUNK)],
            wsem.at[b],
        )

    def wait_gather(g, b):
        pltpu.make_async_copy(
            table_hbm.at[idx_v.at[pl.ds(g * CHUNK, CHUNK)]],
            rows_v.at[b],
            gsem.at[b],
        ).wait()

    def wait_write(g, b):
        pltpu.make_async_copy(
            rows_v.at[b],
            out_hbm.at[pl.ds(base + g * CHUNK, CHUNK)],
            wsem.at[b],
        ).wait()

    for j in range(NBUF):
        start_gather(j, j)
        if j >= 1:
            wait_gather(j - 1, j - 1)
            start_write(j - 1, j - 1)

    @pl.loop(NBUF, N_CHUNKS, step=NBUF)
    def _(g0):
        for j in range(NBUF):
            g = g0 + j
            b = j
            p = (j - 1) % NBUF
            wait_write(g - NBUF, b)
            start_gather(g, b)
            wait_gather(g - 1, p)
            start_write(g - 1, p)

    last = N_CHUNKS - 1
    bl = last % NBUF
    wait_gather(last, bl)
    start_write(last, bl)
    for j in range(NBUF):
        g = N_CHUNKS - NBUF + j
        wait_write(g, g % NBUF)


def kernel(X, pos_embed_weight):
    flat_idx = X.reshape(-1).astype(jnp.int32)
    out = _gather_kernel(flat_idx, pos_embed_weight)
    return out.reshape(X.shape + (D,))
